# baseline (speedup 1.0000x reference)
import jax
import jax.numpy as jnp
from jax import lax
from jax.experimental import pallas as pl
from jax.experimental.pallas import tpu as pltpu
from jax.experimental.pallas import tpu_sc as plsc

N_NODES = 10000
N_EDGES = 320000
D_FEAT = 128

NC = 2
NS = 16
NW = NC * NS
EDGES_PER_WORKER = N_EDGES // NW
CHUNK = 50
NCHUNKS = EDGES_PER_WORKER // CHUNK
IDXBLK = 25
NBLK = NCHUNKS // IDXBLK
NBUF = 4
ZW = 80
N_PAD = 10240
ROWS_PER_TILE = N_PAD // NS


def _sc_body(x_hbm, u_hbm, v_hbm, out_hbm, u_v, v_v, rows_v, acc,
             gsem, asem, isem):
    cid = lax.axis_index("c")
    sid = lax.axis_index("s")
    wid = sid * NC + cid

    zero16 = jnp.zeros((16,), jnp.float32)

    @pl.loop(0, ZW * (D_FEAT // 16))
    def _zero(i):
        r = i // (D_FEAT // 16)
        c = (i % (D_FEAT // 16)) * 16
        rows_v[0, r, pl.ds(c, 16)] = zero16

    pltpu.sync_copy(u_hbm.at[wid, 0], u_v.at[0])
    pltpu.sync_copy(v_hbm.at[wid, 0], v_v.at[0])

    base = sid * ROWS_PER_TILE
    for k in range(ROWS_PER_TILE // ZW):
        pltpu.sync_copy(
            rows_v.at[0, pl.ds(0, ZW)], acc.at[pl.ds(base + k * ZW, ZW)]
        )
    plsc.subcore_barrier()

    pltpu.async_copy(x_hbm.at[u_v.at[0, 0]], rows_v.at[0], gsem.at[0])
    pltpu.async_copy(x_hbm.at[u_v.at[0, 1]], rows_v.at[1], gsem.at[1])

    @pl.loop(0, NCHUNKS)
    def _chunk(k):
        p = k % NBUF
        s = (k // IDXBLK) % 2
        j = k % IDXBLK
        kn = k + 2
        bn = kn // IDXBLK
        sn = bn % 2
        jn = kn % IDXBLK

        pltpu.make_async_copy(
            x_hbm.at[u_v.at[s, j]], rows_v.at[p], gsem.at[p]
        ).wait()

        @pl.when(k >= 1)
        def _():
            pltpu.make_async_copy(
                rows_v.at[(k - 1) % NBUF], acc.at[v_v.at[0, 0]],
                asem.at[0],
            ).wait()

        pltpu.async_copy(rows_v.at[p], acc.at[v_v.at[s, j]], asem.at[0],
                         add=True)

        @pl.when(kn < NCHUNKS)
        def _():
            @pl.when(jn == 0)
            def _():
                pltpu.make_async_copy(
                    u_hbm.at[wid, 0], u_v.at[0], isem.at[0]
                ).wait()
                pltpu.make_async_copy(
                    v_hbm.at[wid, 0], v_v.at[0], isem.at[1]
                ).wait()

            pltpu.async_copy(x_hbm.at[u_v.at[sn, jn]], rows_v.at[kn % NBUF],
                             gsem.at[kn % NBUF])

        @pl.when((jn == 2) & (bn + 1 < NBLK))
        def _():
            pltpu.async_copy(u_hbm.at[wid, bn + 1], u_v.at[(bn + 1) % 2],
                             isem.at[0])
            pltpu.async_copy(v_hbm.at[wid, bn + 1], v_v.at[(bn + 1) % 2],
                             isem.at[1])

    pltpu.make_async_copy(
        rows_v.at[(NCHUNKS - 1) % NBUF], acc.at[v_v.at[0, 0]],
        asem.at[0],
    ).wait()
    plsc.subcore_barrier()

    pltpu.sync_copy(
        acc.at[pl.ds(base, ROWS_PER_TILE)],
        out_hbm.at[cid, pl.ds(base, ROWS_PER_TILE)],
    )


_sc_scatter = pl.kernel(
    _sc_body,
    out_type=jax.ShapeDtypeStruct((NC, N_PAD, D_FEAT), jnp.float32),
    mesh=plsc.VectorSubcoreMesh(
        core_axis_name="c", subcore_axis_name="s", num_cores=NC, num_subcores=NS
    ),
    scratch_types=[
        pltpu.VMEM((2, IDXBLK, CHUNK), jnp.int32),
        pltpu.VMEM((2, IDXBLK, CHUNK), jnp.int32),
        pltpu.VMEM((NBUF, CHUNK, D_FEAT), jnp.float32),
        pltpu.VMEM_SHARED((N_PAD, D_FEAT), jnp.float32),
        pltpu.SemaphoreType.DMA((NBUF,)),
        pltpu.SemaphoreType.DMA((NBUF,)),
        pltpu.SemaphoreType.DMA((2,)),
    ],
)


def _combine_body(p_ref, o_ref):
    o_ref[...] = p_ref[0, :N_NODES] + p_ref[1, :N_NODES]


def _combine(partials):
    return pl.pallas_call(
        _combine_body,
        out_shape=jax.ShapeDtypeStruct((N_NODES, D_FEAT), jnp.float32),
    )(partials)


@jax.jit
def kernel(x, u, v):
    u4 = u.reshape(NW, NBLK, IDXBLK, CHUNK)
    v4 = v.reshape(NW, NBLK, IDXBLK, CHUNK)
    partials = _sc_scatter(x, u4, v4)
    return _combine(partials)


# speedup vs baseline: 1.1740x; 1.1740x over previous
import jax
import jax.numpy as jnp
from jax import lax
from jax.experimental import pallas as pl
from jax.experimental.pallas import tpu as pltpu
from jax.experimental.pallas import tpu_sc as plsc

N_NODES = 10000
N_EDGES = 320000
D_FEAT = 128

NC = 2
NS = 16
NW = NC * NS
EDGES_PER_WORKER = N_EDGES // NW
CHUNK = 80
NCHUNKS = EDGES_PER_WORKER // CHUNK
IDXBLK = 25
NBLK = NCHUNKS // IDXBLK
NBUF = 3
ZW = 80
N_PAD = 10240
ROWS_PER_TILE = N_PAD // NS


def _sc_body(x_hbm, u_hbm, v_hbm, out_hbm, u_v, v_v, rows_v, acc,
             gsem, asem, isem):
    cid = lax.axis_index("c")
    sid = lax.axis_index("s")
    wid = sid * NC + cid

    zero16 = jnp.zeros((16,), jnp.float32)

    @pl.loop(0, ZW * (D_FEAT // 16))
    def _zero(i):
        r = i // (D_FEAT // 16)
        c = (i % (D_FEAT // 16)) * 16
        rows_v[0, r, pl.ds(c, 16)] = zero16

    pltpu.sync_copy(u_hbm.at[wid, 0], u_v.at[0])
    pltpu.sync_copy(v_hbm.at[wid, 0], v_v.at[0])

    base = sid * ROWS_PER_TILE
    for k in range(ROWS_PER_TILE // ZW):
        pltpu.sync_copy(
            rows_v.at[0, pl.ds(0, ZW)], acc.at[pl.ds(base + k * ZW, ZW)]
        )
    plsc.subcore_barrier()

    pltpu.async_copy(x_hbm.at[u_v.at[0, 0]], rows_v.at[0], gsem.at[0])
    pltpu.async_copy(x_hbm.at[u_v.at[0, 1]], rows_v.at[1], gsem.at[1])

    @pl.loop(0, NCHUNKS)
    def _chunk(k):
        p = k % NBUF
        s = (k // IDXBLK) % 2
        j = k % IDXBLK
        kn = k + 2
        bn = kn // IDXBLK
        sn = bn % 2
        jn = kn % IDXBLK

        pltpu.make_async_copy(
            x_hbm.at[u_v.at[s, j]], rows_v.at[p], gsem.at[p]
        ).wait()

        @pl.when(k >= 1)
        def _():
            pltpu.make_async_copy(
                rows_v.at[(k - 1) % NBUF], acc.at[v_v.at[0, 0]],
                asem.at[0],
            ).wait()

        pltpu.async_copy(rows_v.at[p], acc.at[v_v.at[s, j]], asem.at[0],
                         add=True)

        @pl.when(kn < NCHUNKS)
        def _():
            @pl.when(jn == 0)
            def _():
                pltpu.make_async_copy(
                    u_hbm.at[wid, 0], u_v.at[0], isem.at[0]
                ).wait()
                pltpu.make_async_copy(
                    v_hbm.at[wid, 0], v_v.at[0], isem.at[1]
                ).wait()

            pltpu.async_copy(x_hbm.at[u_v.at[sn, jn]], rows_v.at[kn % NBUF],
                             gsem.at[kn % NBUF])

        @pl.when((jn == 2) & (bn + 1 < NBLK))
        def _():
            pltpu.async_copy(u_hbm.at[wid, bn + 1], u_v.at[(bn + 1) % 2],
                             isem.at[0])
            pltpu.async_copy(v_hbm.at[wid, bn + 1], v_v.at[(bn + 1) % 2],
                             isem.at[1])

    pltpu.make_async_copy(
        rows_v.at[(NCHUNKS - 1) % NBUF], acc.at[v_v.at[0, 0]],
        asem.at[0],
    ).wait()
    plsc.subcore_barrier()

    pltpu.sync_copy(
        acc.at[pl.ds(base, ROWS_PER_TILE)],
        out_hbm.at[cid, pl.ds(base, ROWS_PER_TILE)],
    )


_sc_scatter = pl.kernel(
    _sc_body,
    out_type=jax.ShapeDtypeStruct((NC, N_PAD, D_FEAT), jnp.float32),
    mesh=plsc.VectorSubcoreMesh(
        core_axis_name="c", subcore_axis_name="s", num_cores=NC, num_subcores=NS
    ),
    scratch_types=[
        pltpu.VMEM((2, IDXBLK, CHUNK), jnp.int32),
        pltpu.VMEM((2, IDXBLK, CHUNK), jnp.int32),
        pltpu.VMEM((NBUF, CHUNK, D_FEAT), jnp.float32),
        pltpu.VMEM_SHARED((N_PAD, D_FEAT), jnp.float32),
        pltpu.SemaphoreType.DMA((NBUF,)),
        pltpu.SemaphoreType.DMA((NBUF,)),
        pltpu.SemaphoreType.DMA((2,)),
    ],
)


def _combine_body(p_ref, o_ref):
    o_ref[...] = p_ref[0, :N_NODES] + p_ref[1, :N_NODES]


def _combine(partials):
    return pl.pallas_call(
        _combine_body,
        out_shape=jax.ShapeDtypeStruct((N_NODES, D_FEAT), jnp.float32),
    )(partials)


@jax.jit
def kernel(x, u, v):
    u4 = u.reshape(NW, NBLK, IDXBLK, CHUNK)
    v4 = v.reshape(NW, NBLK, IDXBLK, CHUNK)
    partials = _sc_scatter(x, u4, v4)
    return _combine(partials)


# speedup vs baseline: 1.1861x; 1.0103x over previous
import jax
import jax.numpy as jnp
from jax import lax
from jax.experimental import pallas as pl
from jax.experimental.pallas import tpu as pltpu
from jax.experimental.pallas import tpu_sc as plsc

N_NODES = 10000
N_EDGES = 320000
D_FEAT = 128

NC = 2
NS = 16
NW = NC * NS
EDGES_PER_WORKER = N_EDGES // NW
CHUNK = 80
NCHUNKS = EDGES_PER_WORKER // CHUNK
IDXBLK = 25
NBLK = NCHUNKS // IDXBLK
NBUF = 3
ZW = 80
N_PAD = 10240
ROWS_PER_TILE = N_PAD // NS


def _sc_body(x_hbm, u_hbm, v_hbm, out_hbm, u_v, v_v, rows_v, acc,
             gsem, asem, isem):
    cid = lax.axis_index("c")
    sid = lax.axis_index("s")
    wid = sid * NC + cid

    pltpu.sync_copy(u_hbm.at[wid, 0], u_v.at[0])
    pltpu.sync_copy(v_hbm.at[wid, 0], v_v.at[0])
    pltpu.async_copy(x_hbm.at[u_v.at[0, 0]], rows_v.at[0], gsem.at[0])
    pltpu.async_copy(x_hbm.at[u_v.at[0, 1]], rows_v.at[1], gsem.at[1])

    zero16 = jnp.zeros((16,), jnp.float32)

    @pl.loop(0, ZW * (D_FEAT // 16))
    def _zero(i):
        r = i // (D_FEAT // 16)
        c = (i % (D_FEAT // 16)) * 16
        rows_v[2, r, pl.ds(c, 16)] = zero16

    base = sid * ROWS_PER_TILE
    for k in range(ROWS_PER_TILE // ZW):
        pltpu.sync_copy(
            rows_v.at[2, pl.ds(0, ZW)], acc.at[pl.ds(base + k * ZW, ZW)]
        )
    plsc.subcore_barrier()

    @pl.loop(0, NCHUNKS)
    def _chunk(k):
        p = k % NBUF
        s = (k // IDXBLK) % 2
        j = k % IDXBLK
        kn = k + 2
        bn = kn // IDXBLK
        sn = bn % 2
        jn = kn % IDXBLK

        pltpu.make_async_copy(
            x_hbm.at[u_v.at[s, j]], rows_v.at[p], gsem.at[p]
        ).wait()

        @pl.when(k >= 1)
        def _():
            pltpu.make_async_copy(
                rows_v.at[(k - 1) % NBUF], acc.at[v_v.at[0, 0]],
                asem.at[0],
            ).wait()

        pltpu.async_copy(rows_v.at[p], acc.at[v_v.at[s, j]], asem.at[0],
                         add=True)

        @pl.when(kn < NCHUNKS)
        def _():
            @pl.when(jn == 0)
            def _():
                pltpu.make_async_copy(
                    u_hbm.at[wid, 0], u_v.at[0], isem.at[0]
                ).wait()
                pltpu.make_async_copy(
                    v_hbm.at[wid, 0], v_v.at[0], isem.at[1]
                ).wait()

            pltpu.async_copy(x_hbm.at[u_v.at[sn, jn]], rows_v.at[kn % NBUF],
                             gsem.at[kn % NBUF])

        @pl.when((jn == 2) & (bn + 1 < NBLK))
        def _():
            pltpu.async_copy(u_hbm.at[wid, bn + 1], u_v.at[(bn + 1) % 2],
                             isem.at[0])
            pltpu.async_copy(v_hbm.at[wid, bn + 1], v_v.at[(bn + 1) % 2],
                             isem.at[1])

    pltpu.make_async_copy(
        rows_v.at[(NCHUNKS - 1) % NBUF], acc.at[v_v.at[0, 0]],
        asem.at[0],
    ).wait()
    plsc.subcore_barrier()

    pltpu.sync_copy(
        acc.at[pl.ds(base, ROWS_PER_TILE)],
        out_hbm.at[cid, pl.ds(base, ROWS_PER_TILE)],
    )


_sc_scatter = pl.kernel(
    _sc_body,
    out_type=jax.ShapeDtypeStruct((NC, N_PAD, D_FEAT), jnp.float32),
    mesh=plsc.VectorSubcoreMesh(
        core_axis_name="c", subcore_axis_name="s", num_cores=NC, num_subcores=NS
    ),
    scratch_types=[
        pltpu.VMEM((2, IDXBLK, CHUNK), jnp.int32),
        pltpu.VMEM((2, IDXBLK, CHUNK), jnp.int32),
        pltpu.VMEM((NBUF, CHUNK, D_FEAT), jnp.float32),
        pltpu.VMEM_SHARED((N_PAD, D_FEAT), jnp.float32),
        pltpu.SemaphoreType.DMA((NBUF,)),
        pltpu.SemaphoreType.DMA((NBUF,)),
        pltpu.SemaphoreType.DMA((2,)),
    ],
)


def _combine_body(p_ref, o_ref):
    o_ref[...] = p_ref[0, :N_NODES] + p_ref[1, :N_NODES]


def _combine(partials):
    return pl.pallas_call(
        _combine_body,
        out_shape=jax.ShapeDtypeStruct((N_NODES, D_FEAT), jnp.float32),
    )(partials)


@jax.jit
def kernel(x, u, v):
    u4 = u.reshape(NW, NBLK, IDXBLK, CHUNK)
    v4 = v.reshape(NW, NBLK, IDXBLK, CHUNK)
    partials = _sc_scatter(x, u4, v4)
    return _combine(partials)
